# baseline (speedup 1.0000x reference)
import functools

import jax
import jax.numpy as jnp
from jax import lax
from jax.experimental import pallas as pl
from jax.experimental.pallas import tpu as pltpu
from jax.experimental.pallas import tpu_sc as plsc

N = 10000
E = 320000
D = 128
AVG_D_LOG = 3.4965

NC = 2
NS = 16
NW = NC * NS
NPW = 320
NPW1 = NPW + 1
NPAD = NW * NPW
CE = 3200
NCHUNK = E // CE
HD = D // 2
GB = 32
PCAP = CE + 6 * GB

_BIG = 3.0e38


def _sc_body(hnF, srcI, dstI, S_o, Q_o, M_o, N_o, C_o,
             accS, accQ, accM, accN, accC, srcv, dstv, pendS, pendD, rows,
             sem0, sem1):
  cid = lax.axis_index("c")
  sid = lax.axis_index("s")
  wid = sid * NC + cid
  lo = (wid * NPW).astype(jnp.int32)

  iota = lax.iota(jnp.int32, 16)
  zero16 = jnp.zeros((16,), jnp.float32)
  one16 = jnp.ones((16,), jnp.float32)
  neg16 = jnp.full((16,), -_BIG, jnp.float32)
  pos16 = jnp.full((16,), _BIG, jnp.float32)
  zero16i = jnp.zeros((16,), jnp.int32)
  sent16 = jnp.full((16,), NPW, jnp.int32)
  sems = (sem0, sem1)

  def initp(i, _):
    pendS[pl.ds(i * 16, 16)] = zero16i
    pendD[pl.ds(i * 16, 16)] = sent16
    return 0
  lax.fori_loop(0, PCAP // 16, initp, 0)

  for p in range(2):

    def init_body(i, _):
      sl = pl.ds(i * 16, 16)
      accS[sl] = zero16
      accQ[sl] = zero16
      accM[sl] = neg16
      accN[sl] = pos16
      return 0
    lax.fori_loop(0, NPW1 * HD // 16, init_body, 0)
    if p == 0:
      def initc(i, _):
        accC[pl.ds(i * 16, 16)] = zero16
        return 0
      lax.fori_loop(0, NPW1, initc, 0)

    def chunk_body(ci, _):
      base_e = ci * CE
      pltpu.sync_copy(srcI.at[pl.ds(base_e, CE)], srcv)
      pltpu.sync_copy(dstI.at[pl.ds(base_e, CE)], dstv)

      def filt(i, cnt):
        sl = pl.ds(i * 16, 16)
        d = dstv[sl]
        sv = srcv[sl]
        m = (d >= lo) & (d < lo + NPW)
        csum = plsc.cumsum(m.astype(jnp.int32))
        pos = cnt + csum - 1
        plsc.store_scatter(pendS, [pos], sv, mask=m)
        plsc.store_scatter(pendD, [pos], d - lo, mask=m)
        return cnt + csum[15]
      npend = lax.fori_loop(0, CE // 16, filt, 0, unroll=4)

      for k in range(6):
        pendD[pl.ds(npend + 16 * k, 16)] = sent16
        pendS[pl.ds(npend + 16 * k, 16)] = zero16i
      nb = npend // GB + 1

      def start_gather(b, batch):
        for k0 in range(0, GB, 16):
          vec = pendS[pl.ds(batch * GB + k0, 16)]
          for k in range(16):
            pltpu.async_copy(hnF.at[vec[k]], rows.at[b, k0 + k], sems[b])

      def drain(b):
        pltpu.make_async_copy(hnF.at[pl.ds(0, GB)], rows.at[b],
                              sems[b]).wait()

      def process(b, batch):
        for k in range(GB):
          e = batch * GB + k
          dlv = plsc.load_gather(pendD, [jnp.full((16,), e, jnp.int32)])
          base = dlv * HD + iota
          for ch in range(HD // 16):
            idx = base + (ch * 16)
            sv = rows[b, k, pl.ds(p * HD + ch * 16, 16)]
            plsc.addupdate_scatter(accS, [idx], sv)
            plsc.addupdate_scatter(accQ, [idx], sv * sv)
            mv = plsc.load_gather(accM, [idx])
            plsc.store_scatter(accM, [idx], jnp.maximum(mv, sv))
            nv = plsc.load_gather(accN, [idx])
            plsc.store_scatter(accN, [idx], jnp.minimum(nv, sv))
          if p == 0:
            cidx = dlv * 16 + iota
            cv = plsc.load_gather(accC, [cidx])
            plsc.store_scatter(accC, [cidx], cv + one16)

      start_gather(0, 0)
      def pipe(j, _):
        start_gather(1, 2 * j + 1)
        drain(0)
        process(0, 2 * j)
        start_gather(0, 2 * j + 2)
        drain(1)
        process(1, 2 * j + 1)
        return 0
      lax.fori_loop(0, (nb + 1) // 2, pipe, 0)
      drain(0)
      return 0
    lax.fori_loop(0, NCHUNK, chunk_body, 0)

    off = lo * HD + p * (NPAD * HD)
    pltpu.sync_copy(accS.at[pl.ds(0, NPW * HD)], S_o.at[pl.ds(off, NPW * HD)])
    pltpu.sync_copy(accQ.at[pl.ds(0, NPW * HD)], Q_o.at[pl.ds(off, NPW * HD)])
    pltpu.sync_copy(accM.at[pl.ds(0, NPW * HD)], M_o.at[pl.ds(off, NPW * HD)])
    pltpu.sync_copy(accN.at[pl.ds(0, NPW * HD)], N_o.at[pl.ds(off, NPW * HD)])
    if p == 0:
      pltpu.sync_copy(accC.at[pl.ds(0, NPW * 16)],
                      C_o.at[pl.ds(lo * 16, NPW * 16)])


@jax.jit
def _sc_stats(hnF, src, dst):
  mesh = plsc.VectorSubcoreMesh(core_axis_name="c", subcore_axis_name="s")
  f32 = jnp.float32
  out_type = (
      jax.ShapeDtypeStruct((2 * NPAD * HD,), f32),
      jax.ShapeDtypeStruct((2 * NPAD * HD,), f32),
      jax.ShapeDtypeStruct((2 * NPAD * HD,), f32),
      jax.ShapeDtypeStruct((2 * NPAD * HD,), f32),
      jax.ShapeDtypeStruct((NPAD * 16,), f32),
  )
  scratch = [
      pltpu.VMEM((NPW1 * HD,), f32),
      pltpu.VMEM((NPW1 * HD,), f32),
      pltpu.VMEM((NPW1 * HD,), f32),
      pltpu.VMEM((NPW1 * HD,), f32),
      pltpu.VMEM((NPW1 * 16,), f32),
      pltpu.VMEM((CE,), jnp.int32),
      pltpu.VMEM((CE,), jnp.int32),
      pltpu.VMEM((PCAP,), jnp.int32),
      pltpu.VMEM((PCAP,), jnp.int32),
      pltpu.VMEM((2, GB, D), f32),
      pltpu.SemaphoreType.DMA,
      pltpu.SemaphoreType.DMA,
  ]
  k = pl.kernel(_sc_body, out_type=out_type, mesh=mesh,
                scratch_types=scratch,
                compiler_params=pltpu.CompilerParams(
                    needs_layout_passes=False,
                    use_tc_tiling_on_sc=False))
  return k(hnF, src, dst)


def _pre_body(h_ref, norm_ref, hn_ref, hnf_ref):
  hn = h_ref[...] * norm_ref[...]
  hn_ref[0, :, :] = hn[:, :HD]
  hn_ref[1, :, :] = hn[:, HD:]
  hnf_ref[...] = hn


@jax.jit
def _pre(h, norm):
  return pl.pallas_call(
      _pre_body,
      out_shape=(jax.ShapeDtypeStruct((2, N, HD), jnp.float32),
                 jax.ShapeDtypeStruct((N, D), jnp.float32)),
  )(h, norm)


RB = 2000
NB = N // RB


def _tower(hn, norm, deg, S, Q, M, Nn):
  degc = jnp.maximum(deg, 1.0)
  has = deg > 0.0
  amp = jnp.log(deg + 1.0) / AVG_D_LOG
  att = AVG_D_LOG / jnp.log(degc + 1.0)
  scale = (1.0 + amp + att) / 13.0
  mean = (S + deg * hn) / degc
  es = S / degc
  var = jnp.maximum(Q / degc - es * es, 0.0)
  std = jnp.sqrt(var + 1e-5)
  mx = jnp.where(has, M + hn, 0.0)
  mn = jnp.where(has, Nn + hn, 0.0)
  return hn / 13.0 + (mean + mx + mn + std) * scale


def _post_body(hn_ref, norm_ref, cnt_ref, S_ref, Q_ref, M_ref, N_ref,
               gt_ref, bt_ref, gl_ref, bl_ref, out_ref, mom_ref):
  ph = pl.program_id(0)
  rb = pl.program_id(1)
  norm = norm_ref[...]
  deg = cnt_ref[...]

  @pl.when(jnp.logical_and(ph == 0, rb == 0))
  def _():
    mom_ref[...] = jnp.zeros_like(mom_ref)

  halves = []
  for hh in range(2):
    t = _tower(hn_ref[hh], norm, deg, S_ref[hh], Q_ref[hh], M_ref[hh],
               N_ref[hh])
    halves.append(t)

  @pl.when(ph == 0)
  def _():
    n1 = norm
    n2 = norm * norm
    sn = jnp.sum(n1) * jnp.ones((1, HD), jnp.float32)
    sn2 = jnp.sum(n2) * jnp.ones((1, HD), jnp.float32)
    for hh in range(2):
      t = halves[hh]
      mom_ref[hh, 0:1, :] += jnp.sum(t, 0, keepdims=True)
      mom_ref[hh, 1:2, :] += jnp.sum(t * t, 0, keepdims=True)
      mom_ref[hh, 2:3, :] += jnp.sum(t * n1, 0, keepdims=True)
      mom_ref[hh, 3:4, :] += jnp.sum(t * t * n2, 0, keepdims=True)
      mom_ref[hh, 4:5, :] += jnp.sum(t * n2, 0, keepdims=True)
      mom_ref[hh, 5:6, :] += sn
      mom_ref[hh, 6:7, :] += sn2
    out_ref[...] = jnp.zeros_like(out_ref)

  @pl.when(ph == 1)
  def _():
    outs = []
    fn = jnp.float32(N)
    for hh in range(2):
      t = halves[hh]
      t1 = mom_ref[hh, 0:1, :]
      t2 = mom_ref[hh, 1:2, :]
      p1 = mom_ref[hh, 2:3, :]
      p2 = mom_ref[hh, 3:4, :]
      p3 = mom_ref[hh, 4:5, :]
      sn = mom_ref[hh, 5:6, :]
      sn2 = mom_ref[hh, 6:7, :]
      mu1 = t1 / fn
      v1 = jnp.maximum(t2 / fn - mu1 * mu1, 0.0)
      a = gt_ref[hh:hh + 1, :] / jnp.sqrt(v1 + 1e-5)
      c = bt_ref[hh:hh + 1, :] - mu1 * a
      sy = a * p1 + c * sn
      sy2 = a * a * p2 + 2.0 * a * c * p3 + c * c * sn2
      mu2 = sy / fn
      v2 = jnp.maximum(sy2 / fn - mu2 * mu2, 0.0)
      y = (a * t + c) * norm
      outs.append((y - mu2) / jnp.sqrt(v2 + 1e-5) * gl_ref[hh:hh + 1, :]
                  + bl_ref[hh:hh + 1, :])
    out_ref[...] = jnp.concatenate(outs, axis=1)


@jax.jit
def _post(hn2, norm, cnt, S, Q, M, Nn, gt, bt, gl, bl):
  stat_spec = pl.BlockSpec((2, RB, HD), lambda ph, rb: (0, rb, 0))
  vec_spec = pl.BlockSpec((RB, 1), lambda ph, rb: (rb, 0))
  gb_spec = pl.BlockSpec((2, HD), lambda ph, rb: (0, 0))
  return pl.pallas_call(
      _post_body,
      grid=(2, NB),
      in_specs=[stat_spec, vec_spec, vec_spec,
                stat_spec, stat_spec, stat_spec, stat_spec,
                gb_spec, gb_spec, gb_spec, gb_spec],
      out_specs=pl.BlockSpec((RB, D), lambda ph, rb: (rb, 0)),
      out_shape=jax.ShapeDtypeStruct((N, D), jnp.float32),
      scratch_shapes=[pltpu.VMEM((2, 8, HD), jnp.float32)],
  )(hn2, norm, cnt, S, Q, M, Nn, gt, bt, gl, bl)


def kernel(h, edge_index, e, norm, gamma_t, beta_t, gamma_l, beta_l):
  src = edge_index[0]
  dst = edge_index[1]
  hn2, hnf = _pre(h, norm)
  S, Q, M, Nn, cnt = _sc_stats(hnf, src, dst)
  S = S.reshape(2, NPAD, HD)
  Q = Q.reshape(2, NPAD, HD)
  M = M.reshape(2, NPAD, HD)
  Nn = Nn.reshape(2, NPAD, HD)
  cnt2 = cnt.reshape(NPAD, 16)[:N, :1]
  out = _post(hn2, norm, cnt2, S, Q, M, Nn,
              gamma_t.reshape(2, HD), beta_t.reshape(2, HD),
              gamma_l.reshape(2, HD), beta_l.reshape(2, HD))
  return out


# speedup vs baseline: 5.5425x; 5.5425x over previous
import jax
import jax.numpy as jnp
from jax import lax
from jax.experimental import pallas as pl
from jax.experimental.pallas import tpu as pltpu
from jax.experimental.pallas import tpu_sc as plsc

N = 10000
E = 320000
D = 128
AVG_D_LOG = 3.4965

NC = 2
NS = 16
NW = NC * NS
NPW = 320
NPW1 = NPW + 1
NPAD = NW * NPW
CE = 1280
NCHUNK = E // CE
HD = D // 2
GB = 8
PCAP = CE + 32

_BIG = 3.0e38


def _sc_body(hn2, srcI, dstI, S_o, Q_o, M_o, N_o, C_o,
             accS, accQ, accM, accN, accC, srcv, dstv, pendS, pendD, rows,
             hnS, sem0, sem1):
  cid = lax.axis_index("c")
  sid = lax.axis_index("s")
  wid = sid * NC + cid
  lo = (wid * NPW).astype(jnp.int32)

  iota = lax.iota(jnp.int32, 16)
  zero16 = jnp.zeros((16,), jnp.float32)
  one16 = jnp.ones((16,), jnp.float32)
  neg16 = jnp.full((16,), -_BIG, jnp.float32)
  pos16 = jnp.full((16,), _BIG, jnp.float32)
  zero16i = jnp.zeros((16,), jnp.int32)
  sent16 = jnp.full((16,), NPW, jnp.int32)
  lane0 = iota == 0
  sems = (sem0, sem1)
  stripe = NPAD // NS

  def initp(i, _):
    pendS[pl.ds(i * 16, 16)] = zero16i
    pendD[pl.ds(i * 16, 16)] = sent16
    return 0
  lax.fori_loop(0, PCAP // 16, initp, 0)

  for p in range(2):
    pltpu.sync_copy(hn2.at[p, pl.ds(sid * stripe, stripe)],
                    hnS.at[pl.ds(sid * stripe, stripe)])
    plsc.subcore_barrier()

    def init_body(i, _):
      sl = pl.ds(i * 16, 16)
      accS[sl] = zero16
      accQ[sl] = zero16
      accM[sl] = neg16
      accN[sl] = pos16
      return 0
    lax.fori_loop(0, NPW1 * HD // 16, init_body, 0)
    if p == 0:
      def initc(i, _):
        accC[pl.ds(i * 16, 16)] = zero16
        return 0
      lax.fori_loop(0, (NPW1 + 15) // 16, initc, 0)

    def chunk_body(ci, _):
      base_e = ci * CE
      pltpu.sync_copy(srcI.at[pl.ds(base_e, CE)], srcv)
      pltpu.sync_copy(dstI.at[pl.ds(base_e, CE)], dstv)

      def filt(i, cnt):
        sl = pl.ds(i * 16, 16)
        d = dstv[sl]
        sv = srcv[sl]
        m = (d >= lo) & (d < lo + NPW)
        csum = plsc.cumsum(m.astype(jnp.int32))
        pos = cnt + csum - 1
        plsc.store_scatter(pendS, [pos], sv, mask=m)
        plsc.store_scatter(pendD, [pos], d - lo, mask=m)
        return cnt + csum[15]
      npend = lax.fori_loop(0, CE // 16, filt, 0, unroll=4)

      for k in range(2):
        pendD[pl.ds(npend + 16 * k, 16)] = sent16
        pendS[pl.ds(npend + 16 * k, 16)] = zero16i
      nb = npend // GB + 1

      def start_gather(b, batch):
        return pltpu.async_copy(hnS.at[pendS.at[pl.ds(batch * GB, GB)]],
                                rows.at[b], sems[b])

      def drain(b):
        pltpu.make_async_copy(hnS.at[pl.ds(0, GB)], rows.at[b],
                              sems[b]).wait()

      def process(b, batch):
        for k in range(GB):
          e = batch * GB + k
          dlv = plsc.load_gather(pendD, [jnp.full((16,), e, jnp.int32)])
          base = dlv * HD + iota
          for ch in range(HD // 16):
            idx = base + (ch * 16)
            sv = rows[b, k, pl.ds(ch * 16, 16)]
            plsc.addupdate_scatter(accS, [idx], sv)
            plsc.addupdate_scatter(accQ, [idx], sv * sv)
            mv = plsc.load_gather(accM, [idx])
            plsc.store_scatter(accM, [idx], jnp.maximum(mv, sv))
            nv = plsc.load_gather(accN, [idx])
            plsc.store_scatter(accN, [idx], jnp.minimum(nv, sv))
          if p == 0:
            plsc.addupdate_scatter(accC, [dlv], one16, mask=lane0)

      start_gather(0, 0)
      def pipe(j, _):
        start_gather(1, 2 * j + 1)
        drain(0)
        process(0, 2 * j)
        start_gather(0, 2 * j + 2)
        drain(1)
        process(1, 2 * j + 1)
        return 0
      lax.fori_loop(0, (nb + 1) // 2, pipe, 0)
      drain(0)
      return 0
    lax.fori_loop(0, NCHUNK, chunk_body, 0)

    off = lo * HD + p * (NPAD * HD)
    pltpu.sync_copy(accS.at[pl.ds(0, NPW * HD)], S_o.at[pl.ds(off, NPW * HD)])
    pltpu.sync_copy(accQ.at[pl.ds(0, NPW * HD)], Q_o.at[pl.ds(off, NPW * HD)])
    pltpu.sync_copy(accM.at[pl.ds(0, NPW * HD)], M_o.at[pl.ds(off, NPW * HD)])
    pltpu.sync_copy(accN.at[pl.ds(0, NPW * HD)], N_o.at[pl.ds(off, NPW * HD)])
    if p == 0:
      pltpu.sync_copy(accC.at[pl.ds(0, NPW)], C_o.at[pl.ds(lo, NPW)])


@jax.jit
def _sc_stats(hn2, src, dst):
  mesh = plsc.VectorSubcoreMesh(core_axis_name="c", subcore_axis_name="s")
  f32 = jnp.float32
  out_type = (
      jax.ShapeDtypeStruct((2 * NPAD * HD,), f32),
      jax.ShapeDtypeStruct((2 * NPAD * HD,), f32),
      jax.ShapeDtypeStruct((2 * NPAD * HD,), f32),
      jax.ShapeDtypeStruct((2 * NPAD * HD,), f32),
      jax.ShapeDtypeStruct((NPAD,), f32),
  )
  scratch = [
      pltpu.VMEM((NPW1 * HD,), f32),
      pltpu.VMEM((NPW1 * HD,), f32),
      pltpu.VMEM((NPW1 * HD,), f32),
      pltpu.VMEM((NPW1 * HD,), f32),
      pltpu.VMEM((NPW1 + 15,), f32),
      pltpu.VMEM((CE,), jnp.int32),
      pltpu.VMEM((CE,), jnp.int32),
      pltpu.VMEM((PCAP,), jnp.int32),
      pltpu.VMEM((PCAP,), jnp.int32),
      pltpu.VMEM((2, GB, HD), f32),
      pltpu.VMEM_SHARED((NPAD, HD), f32),
      pltpu.SemaphoreType.DMA,
      pltpu.SemaphoreType.DMA,
  ]
  k = pl.kernel(_sc_body, out_type=out_type, mesh=mesh,
                scratch_types=scratch,
                compiler_params=pltpu.CompilerParams(
                    needs_layout_passes=False,
                    use_tc_tiling_on_sc=False))
  return k(hn2, src, dst)


def _pre_body(h_ref, norm_ref, hn_ref):
  hn = h_ref[...] * norm_ref[...]
  pad = jnp.zeros((NPAD - N, HD), jnp.float32)
  hn_ref[0, :, :] = jnp.concatenate([hn[:, :HD], pad], axis=0)
  hn_ref[1, :, :] = jnp.concatenate([hn[:, HD:], pad], axis=0)


@jax.jit
def _pre(h, norm):
  return pl.pallas_call(
      _pre_body,
      out_shape=jax.ShapeDtypeStruct((2, NPAD, HD), jnp.float32),
  )(h, norm)


RB = 2000
NB = N // RB


def _tower(hn, norm, deg, S, Q, M, Nn):
  degc = jnp.maximum(deg, 1.0)
  has = deg > 0.0
  amp = jnp.log(deg + 1.0) / AVG_D_LOG
  att = AVG_D_LOG / jnp.log(degc + 1.0)
  scale = (1.0 + amp + att) / 13.0
  mean = (S + deg * hn) / degc
  es = S / degc
  var = jnp.maximum(Q / degc - es * es, 0.0)
  std = jnp.sqrt(var + 1e-5)
  mx = jnp.where(has, M + hn, 0.0)
  mn = jnp.where(has, Nn + hn, 0.0)
  return hn / 13.0 + (mean + mx + mn + std) * scale


def _post_body(hn_ref, norm_ref, cnt_ref, S_ref, Q_ref, M_ref, N_ref,
               gt_ref, bt_ref, gl_ref, bl_ref, out_ref, mom_ref):
  ph = pl.program_id(0)
  rb = pl.program_id(1)
  norm = norm_ref[...]
  deg = cnt_ref[...]

  @pl.when(jnp.logical_and(ph == 0, rb == 0))
  def _():
    mom_ref[...] = jnp.zeros_like(mom_ref)

  halves = []
  for hh in range(2):
    t = _tower(hn_ref[hh, :RB, :], norm, deg, S_ref[hh], Q_ref[hh],
               M_ref[hh], N_ref[hh])
    halves.append(t)

  @pl.when(ph == 0)
  def _():
    n1 = norm
    n2 = norm * norm
    sn = jnp.sum(n1) * jnp.ones((1, HD), jnp.float32)
    sn2 = jnp.sum(n2) * jnp.ones((1, HD), jnp.float32)
    for hh in range(2):
      t = halves[hh]
      mom_ref[hh, 0:1, :] += jnp.sum(t, 0, keepdims=True)
      mom_ref[hh, 1:2, :] += jnp.sum(t * t, 0, keepdims=True)
      mom_ref[hh, 2:3, :] += jnp.sum(t * n1, 0, keepdims=True)
      mom_ref[hh, 3:4, :] += jnp.sum(t * t * n2, 0, keepdims=True)
      mom_ref[hh, 4:5, :] += jnp.sum(t * n2, 0, keepdims=True)
      mom_ref[hh, 5:6, :] += sn
      mom_ref[hh, 6:7, :] += sn2
    out_ref[...] = jnp.zeros_like(out_ref)

  @pl.when(ph == 1)
  def _():
    outs = []
    fn = jnp.float32(N)
    for hh in range(2):
      t = halves[hh]
      t1 = mom_ref[hh, 0:1, :]
      t2 = mom_ref[hh, 1:2, :]
      p1 = mom_ref[hh, 2:3, :]
      p2 = mom_ref[hh, 3:4, :]
      p3 = mom_ref[hh, 4:5, :]
      sn = mom_ref[hh, 5:6, :]
      sn2 = mom_ref[hh, 6:7, :]
      mu1 = t1 / fn
      v1 = jnp.maximum(t2 / fn - mu1 * mu1, 0.0)
      a = gt_ref[hh:hh + 1, :] / jnp.sqrt(v1 + 1e-5)
      c = bt_ref[hh:hh + 1, :] - mu1 * a
      sy = a * p1 + c * sn
      sy2 = a * a * p2 + 2.0 * a * c * p3 + c * c * sn2
      mu2 = sy / fn
      v2 = jnp.maximum(sy2 / fn - mu2 * mu2, 0.0)
      y = (a * t + c) * norm
      outs.append((y - mu2) / jnp.sqrt(v2 + 1e-5) * gl_ref[hh:hh + 1, :]
                  + bl_ref[hh:hh + 1, :])
    out_ref[...] = jnp.concatenate(outs, axis=1)


@jax.jit
def _post(hn2, norm, cnt, S, Q, M, Nn, gt, bt, gl, bl):
  stat_spec = pl.BlockSpec((2, RB, HD), lambda ph, rb: (0, rb, 0))
  vec_spec = pl.BlockSpec((RB, 1), lambda ph, rb: (rb, 0))
  gb_spec = pl.BlockSpec((2, HD), lambda ph, rb: (0, 0))
  return pl.pallas_call(
      _post_body,
      grid=(2, NB),
      in_specs=[stat_spec, vec_spec, vec_spec,
                stat_spec, stat_spec, stat_spec, stat_spec,
                gb_spec, gb_spec, gb_spec, gb_spec],
      out_specs=pl.BlockSpec((RB, D), lambda ph, rb: (rb, 0)),
      out_shape=jax.ShapeDtypeStruct((N, D), jnp.float32),
      scratch_shapes=[pltpu.VMEM((2, 8, HD), jnp.float32)],
  )(hn2, norm, cnt, S, Q, M, Nn, gt, bt, gl, bl)


def kernel(h, edge_index, e, norm, gamma_t, beta_t, gamma_l, beta_l):
  src = edge_index[0]
  dst = edge_index[1]
  hn2 = _pre(h, norm)
  S, Q, M, Nn, cnt = _sc_stats(hn2, src, dst)
  S = S.reshape(2, NPAD, HD)
  Q = Q.reshape(2, NPAD, HD)
  M = M.reshape(2, NPAD, HD)
  Nn = Nn.reshape(2, NPAD, HD)
  cnt2 = cnt[:N, None]
  out = _post(hn2, norm, cnt2, S, Q, M, Nn,
              gamma_t.reshape(2, HD), beta_t.reshape(2, HD),
              gamma_l.reshape(2, HD), beta_l.reshape(2, HD))
  return out
